# Initial kernel scaffold; baseline (speedup 1.0000x reference)
import jax
import jax.numpy as jnp
from jax.experimental import pallas as pl


def kernel(x, gate_w, w1, b1, w2, b2):
    raise NotImplementedError("write your pallas kernel here")


# speedup vs baseline: 2.0535x; 2.0535x over previous
import functools

import jax
import jax.numpy as jnp
from jax import lax
from jax.experimental import pallas as pl
from jax.experimental.pallas import tpu as pltpu

E = 8
TOPK = 2
BT = 256


def _moe_dense_kernel(x_ref, gate_ref, w1_ref, b1_ref, w2_ref, b2_ref,
                      out_ref, comb_ref):
    t = pl.program_id(0)
    e = pl.program_id(1)
    f = pl.program_id(2)

    @pl.when(jnp.logical_and(e == 0, f == 0))
    def _router():
        xb = x_ref[...]
        s = lax.dot_general(xb, gate_ref[...],
                            (((1,), (1,)), ((), ())),
                            preferred_element_type=jnp.float32)
        idx1 = jnp.argmax(s, axis=1)
        cols = lax.broadcasted_iota(jnp.int32, s.shape, 1)
        oh1 = (cols == idx1[:, None])
        m1 = jnp.max(s, axis=1, keepdims=True)
        s2 = jnp.where(oh1, -jnp.inf, s)
        idx2 = jnp.argmax(s2, axis=1)
        oh2 = (cols == idx2[:, None])
        m2 = jnp.max(s2, axis=1, keepdims=True)
        e2 = jnp.exp(m2 - m1)
        z = 1.0 + e2
        p1 = 1.0 / z
        p2 = e2 / z
        comb_ref[...] = jnp.where(oh1, p1, 0.0) + jnp.where(oh2, p2, 0.0)
        out_ref[...] = jnp.zeros_like(out_ref)

    xb = x_ref[...]
    w1b = w1_ref[0]
    h = lax.dot_general(xb, w1b, (((1,), (1,)), ((), ())),
                        preferred_element_type=jnp.float32)
    h = h + b1_ref[0]
    h = 0.5 * h * (1.0 + lax.erf(h * 0.7071067811865476))
    w2b = w2_ref[0]
    y = lax.dot_general(h, w2b, (((1,), (1,)), ((), ())),
                        preferred_element_type=jnp.float32)
    comb = comb_ref[...]
    cols_e = lax.broadcasted_iota(jnp.int32, comb.shape, 1)
    ce = jnp.sum(jnp.where(cols_e == e, comb, 0.0), axis=1, keepdims=True)

    @pl.when(f == 0)
    def _bias2():
        out_ref[...] += ce * b2_ref[0]

    out_ref[...] += ce * y


def kernel(x, gate_w, w1, b1, w2, b2):
    b, s, d = x.shape
    xf = x.reshape(-1, d)
    T = xf.shape[0]
    n_exp, f_dim = w1.shape[0], w1.shape[1]
    FB = 1024
    nf = f_dim // FB
    nt = T // BT

    b1r = b1.reshape(n_exp, 1, f_dim)
    b2r = b2.reshape(n_exp, 1, d)

    out = pl.pallas_call(
        _moe_dense_kernel,
        grid=(nt, n_exp, nf),
        in_specs=[
            pl.BlockSpec((BT, d), lambda t, e, f: (t, 0)),
            pl.BlockSpec((n_exp, d), lambda t, e, f: (0, 0)),
            pl.BlockSpec((1, FB, d), lambda t, e, f: (e, f, 0)),
            pl.BlockSpec((1, 1, FB), lambda t, e, f: (e, 0, f)),
            pl.BlockSpec((1, d, FB), lambda t, e, f: (e, 0, f)),
            pl.BlockSpec((1, 1, d), lambda t, e, f: (e, 0, 0)),
        ],
        out_specs=pl.BlockSpec((BT, d), lambda t, e, f: (t, 0)),
        out_shape=jax.ShapeDtypeStruct((T, d), jnp.float32),
        scratch_shapes=[pltpu.VMEM((BT, n_exp), jnp.float32)],
    )(xf, gate_w, w1, b1r, w2, b2r)
    return out.reshape(b, s, d)
